# baseline (speedup 1.0000x reference)
import functools

import jax
import jax.numpy as jnp
from jax import lax
from jax.experimental import pallas as pl
from jax.experimental.pallas import tpu as pltpu
from jax.experimental.pallas import tpu_sc as plsc

NUM_CODES = 8192
CODE_DIM = 256

TN = 256
TK = 2048


def _argmin_body(x_ref, c_ref, idx_ref, minval, minidx):
    k = pl.program_id(1)
    nk = pl.num_programs(1)
    xt = x_ref[...]
    ct = c_ref[...]
    x_sq = jnp.sum(xt * xt, axis=1, keepdims=True)
    c_sq = jnp.sum(ct * ct, axis=1)[None, :]
    mm = lax.dot_general(xt, ct, (((1,), (1,)), ((), ())),
                         preferred_element_type=jnp.float32)
    d2 = (x_sq + c_sq) - 2.0 * mm
    d2 = jnp.maximum(d2, 0.0)
    tmin = jnp.min(d2, axis=1, keepdims=True)
    iota = lax.broadcasted_iota(jnp.int32, d2.shape, 1) + k * TK
    tidx = jnp.min(jnp.where(d2 == tmin, iota, jnp.int32(2**30)),
                   axis=1, keepdims=True)
    dmin = jnp.sqrt(tmin)
    dmin_bf = dmin.astype(jnp.bfloat16).astype(jnp.float32)

    @pl.when(k == 0)
    def _():
        minval[...] = dmin_bf
        minidx[...] = tidx

    @pl.when(k > 0)
    def _():
        better = dmin < minval[...]
        minval[...] = jnp.where(better, dmin_bf, minval[...])
        minidx[...] = jnp.where(better, tidx, minidx[...])

    @pl.when(k == nk - 1)
    def _():
        idx_ref[...] = minidx[...]


def _argmin_call(flat_x, codebook):
    n, d = flat_x.shape
    kk = codebook.shape[0]
    grid = (n // TN, kk // TK)
    return pl.pallas_call(
        _argmin_body,
        grid=grid,
        in_specs=[
            pl.BlockSpec((TN, d), lambda i, j: (i, 0)),
            pl.BlockSpec((TK, d), lambda i, j: (j, 0)),
        ],
        out_specs=pl.BlockSpec((TN, 1), lambda i, j: (i, 0)),
        out_shape=jax.ShapeDtypeStruct((n, 1), jnp.int32),
        scratch_shapes=[
            pltpu.VMEM((TN, 1), jnp.float32),
            pltpu.VMEM((TN, 1), jnp.int32),
        ],
    )(flat_x, codebook)


N_TOKENS = 8192
_NW = 32
_BPW = N_TOKENS // _NW


@functools.cache
def _make_sc_gather():
    @functools.partial(
        pl.kernel,
        out_type=jax.ShapeDtypeStruct((N_TOKENS, CODE_DIM), jnp.float32),
        mesh=plsc.VectorSubcoreMesh(core_axis_name="c", subcore_axis_name="s"),
        scratch_types=[
            pltpu.VMEM((_BPW,), jnp.int32),
            pltpu.VMEM((_BPW, CODE_DIM), jnp.float32),
            pltpu.SemaphoreType.DMA,
        ],
    )
    def _sc_gather(table_hbm, idx_hbm, out_hbm, idx_v, rows_v, sem):
        wid = lax.axis_index("s") * 2 + lax.axis_index("c")
        base = wid * _BPW
        pltpu.sync_copy(idx_hbm.at[pl.ds(base, _BPW)], idx_v)
        pltpu.async_copy(table_hbm.at[idx_v], rows_v, sem).wait()
        pltpu.sync_copy(rows_v, out_hbm.at[pl.ds(base, _BPW)])

    return _sc_gather


def kernel(x, codebook):
    b, s, d = x.shape
    flat_x = x.reshape(-1, d)
    idx = _argmin_call(flat_x, codebook).reshape(-1)
    quantized = _make_sc_gather()(codebook, idx).reshape(b, s, d)
    return quantized, idx.reshape(b, s)


# speedup vs baseline: 1.0064x; 1.0064x over previous
import functools

import jax
import jax.numpy as jnp
from jax import lax
from jax.experimental import pallas as pl
from jax.experimental.pallas import tpu as pltpu
from jax.experimental.pallas import tpu_sc as plsc

NUM_CODES = 8192
CODE_DIM = 256
N_TOKENS = 8192

TN = 256
TK = 2048


def _argmin_body(x_ref, c_ref, idx_ref, xsq_s, csq_s, minval, minidx):
    kk = pl.program_id(0)
    nn = pl.program_id(1)
    nk = pl.num_programs(0)
    rows = pl.ds(nn * TN, TN)
    xt = x_ref[...]
    ct = c_ref[...]

    @pl.when(kk == 0)
    def _():
        xsq_s[rows, :] = jnp.sum(xt * xt, axis=1, keepdims=True)

    @pl.when(nn == 0)
    def _():
        csq_s[...] = jnp.sum(ct * ct, axis=1)[None, :]

    x_sq = xsq_s[rows, :]
    c_sq = csq_s[...]
    mm2 = lax.dot_general(xt * -2.0, ct, (((1,), (1,)), ((), ())),
                          preferred_element_type=jnp.float32)
    d2 = (x_sq + c_sq) + mm2
    tmin = jnp.min(d2, axis=1, keepdims=True)
    iota = lax.broadcasted_iota(jnp.int32, d2.shape, 1) + kk * TK
    tidx = jnp.min(jnp.where(d2 == tmin, iota, jnp.int32(2**30)),
                   axis=1, keepdims=True)
    dmin = jnp.sqrt(jnp.maximum(tmin, 0.0))
    dmin_bf = dmin.astype(jnp.bfloat16).astype(jnp.float32)

    @pl.when(kk == 0)
    def _():
        minval[rows, :] = dmin_bf
        minidx[rows, :] = tidx

    @pl.when(kk > 0)
    def _():
        better = dmin < minval[rows, :]
        minval[rows, :] = jnp.where(better, dmin_bf, minval[rows, :])
        minidx[rows, :] = jnp.where(better, tidx, minidx[rows, :])

    @pl.when(kk == nk - 1)
    def _():
        idx_ref[...] = minidx[rows, :]


def _argmin_call(flat_x, codebook):
    n, d = flat_x.shape
    kk = codebook.shape[0]
    grid = (kk // TK, n // TN)
    return pl.pallas_call(
        _argmin_body,
        grid=grid,
        in_specs=[
            pl.BlockSpec((TN, d), lambda k, i: (i, 0)),
            pl.BlockSpec((TK, d), lambda k, i: (k, 0)),
        ],
        out_specs=pl.BlockSpec((TN, 1), lambda k, i: (i, 0)),
        out_shape=jax.ShapeDtypeStruct((n, 1), jnp.int32),
        scratch_shapes=[
            pltpu.VMEM((n, 1), jnp.float32),
            pltpu.VMEM((1, TK), jnp.float32),
            pltpu.VMEM((n, 1), jnp.float32),
            pltpu.VMEM((n, 1), jnp.int32),
        ],
    )(flat_x, codebook)


_NW = 32
_BPW = N_TOKENS // _NW


@functools.cache
def _make_sc_gather():
    @functools.partial(
        pl.kernel,
        out_type=jax.ShapeDtypeStruct((N_TOKENS, CODE_DIM), jnp.float32),
        mesh=plsc.VectorSubcoreMesh(core_axis_name="c", subcore_axis_name="s"),
        scratch_types=[
            pltpu.VMEM((_BPW,), jnp.int32),
            pltpu.VMEM((_BPW, CODE_DIM), jnp.float32),
            pltpu.SemaphoreType.DMA,
        ],
    )
    def _sc_gather(table_hbm, idx_hbm, out_hbm, idx_v, rows_v, sem):
        wid = lax.axis_index("s") * 2 + lax.axis_index("c")
        base = wid * _BPW
        pltpu.sync_copy(idx_hbm.at[pl.ds(base, _BPW)], idx_v)
        pltpu.async_copy(table_hbm.at[idx_v], rows_v, sem).wait()
        pltpu.sync_copy(rows_v, out_hbm.at[pl.ds(base, _BPW)])

    return _sc_gather


def kernel(x, codebook):
    b, s, d = x.shape
    flat_x = x.reshape(-1, d)
    idx = _argmin_call(flat_x, codebook).reshape(-1)
    quantized = _make_sc_gather()(codebook, idx).reshape(b, s, d)
    return quantized, idx.reshape(b, s)


# speedup vs baseline: 1.1852x; 1.1776x over previous
import functools

import jax
import jax.numpy as jnp
from jax import lax
from jax.experimental import pallas as pl
from jax.experimental.pallas import tpu as pltpu
from jax.experimental.pallas import tpu_sc as plsc

NUM_CODES = 8192
CODE_DIM = 256
N_TOKENS = 8192

TN = 256
TK = 2048


def _argmin_body(x_ref, c_ref, idx_ref, xsq_s, csq_s, minval, minidx):
    kk = pl.program_id(0)
    nn = pl.program_id(1)
    nk = pl.num_programs(0)
    rows = pl.ds(nn * TN, TN)
    xt = x_ref[...]
    ct = c_ref[...]

    @pl.when(kk == 0)
    def _():
        xsq_s[rows, :] = jnp.sum(xt * xt, axis=1, keepdims=True)

    @pl.when(nn == 0)
    def _():
        csq_s[...] = jnp.sum(ct * ct, axis=1)[None, :]

    x_sq = xsq_s[rows, :]
    mm2 = lax.dot_general(xt * -2.0, ct, (((1,), (1,)), ((), ())),
                          preferred_element_type=jnp.float32)
    acc = jnp.full((TN, 128), jnp.inf, dtype=jnp.float32)
    chunk_of = jnp.zeros((TN, 128), dtype=jnp.int32)
    for c in range(TK // 128):
        lo, hi = c * 128, (c + 1) * 128
        d2c = (x_sq + csq_s[:, lo:hi]) + mm2[:, lo:hi]
        better = d2c < acc
        acc = jnp.minimum(acc, d2c)
        chunk_of = jnp.where(better, jnp.int32(c), chunk_of)
    tmin = jnp.min(acc, axis=1, keepdims=True)
    lane = lax.broadcasted_iota(jnp.int32, (TN, 128), 1)
    jidx = chunk_of * 128 + lane
    cand = jnp.where(acc == tmin, jidx, jnp.int32(2**30))
    tidx = (jnp.min(cand, axis=1, keepdims=True)
            + kk * TK)
    dmin = jnp.sqrt(jnp.maximum(tmin, 0.0))
    dmin_bf = dmin.astype(jnp.bfloat16).astype(jnp.float32)

    @pl.when(kk == 0)
    def _():
        minval[rows, :] = dmin_bf
        minidx[rows, :] = tidx

    @pl.when(kk > 0)
    def _():
        better = dmin < minval[rows, :]
        minval[rows, :] = jnp.where(better, dmin_bf, minval[rows, :])
        minidx[rows, :] = jnp.where(better, tidx, minidx[rows, :])

    @pl.when(kk == nk - 1)
    def _():
        idx_ref[...] = minidx[rows, :]


def _argmin_call(flat_x, codebook):
    n, d = flat_x.shape
    kk = codebook.shape[0]
    grid = (kk // TK, n // TN)
    return pl.pallas_call(
        _argmin_body,
        grid=grid,
        in_specs=[
            pl.BlockSpec((TN, d), lambda k, i: (i, 0)),
            pl.BlockSpec((TK, d), lambda k, i: (k, 0)),
        ],
        out_specs=pl.BlockSpec((TN, 1), lambda k, i: (i, 0)),
        out_shape=jax.ShapeDtypeStruct((n, 1), jnp.int32),
        scratch_shapes=[
            pltpu.VMEM((n, 1), jnp.float32),
            pltpu.VMEM((1, TK), jnp.float32),
            pltpu.VMEM((n, 1), jnp.float32),
            pltpu.VMEM((n, 1), jnp.int32),
        ],
    )(flat_x, codebook)


_NW = 32
_BPW = N_TOKENS // _NW


@functools.cache
def _make_sc_gather():
    @functools.partial(
        pl.kernel,
        out_type=jax.ShapeDtypeStruct((N_TOKENS, CODE_DIM), jnp.float32),
        mesh=plsc.VectorSubcoreMesh(core_axis_name="c", subcore_axis_name="s"),
        scratch_types=[
            pltpu.VMEM((_BPW,), jnp.int32),
            pltpu.VMEM((_BPW, CODE_DIM), jnp.float32),
            pltpu.SemaphoreType.DMA,
        ],
    )
    def _sc_gather(table_hbm, idx_hbm, out_hbm, idx_v, rows_v, sem):
        wid = lax.axis_index("s") * 2 + lax.axis_index("c")
        base = wid * _BPW
        pltpu.sync_copy(idx_hbm.at[pl.ds(base, _BPW)], idx_v)
        pltpu.async_copy(table_hbm.at[idx_v], rows_v, sem).wait()
        pltpu.sync_copy(rows_v, out_hbm.at[pl.ds(base, _BPW)])

    return _sc_gather


def kernel(x, codebook):
    b, s, d = x.shape
    flat_x = x.reshape(-1, d)
    idx = _argmin_call(flat_x, codebook).reshape(-1)
    quantized = _make_sc_gather()(codebook, idx).reshape(b, s, d)
    return quantized, idx.reshape(b, s)
